# baseline (speedup 1.0000x reference)
import functools

import jax
import jax.numpy as jnp
from jax import lax
from jax.experimental import pallas as pl
from jax.experimental.pallas import tpu as pltpu
from jax.experimental.pallas import tpu_sc as plsc

_EPS = 1e-12
_NC = 2
_NS = 16
_NW = _NC * _NS
_L = 16
_NBUF = 4


def _rsqrt16(x):
    i = plsc.bitcast(x, jnp.int32)
    y = plsc.bitcast(jnp.int32(0x5F3759DF) - lax.shift_right_logical(i, 1),
                     jnp.float32)
    for _ in range(2):
        y = y * (1.5 - 0.5 * x * y * y)
    return y


def _make_sc_kernel(B, Lseq, E):
    mesh = plsc.VectorSubcoreMesh(core_axis_name="c", subcore_axis_name="s")
    bpw = B // _NW
    splits = [(0, 104), (104, Lseq - 104)]
    assert bpw % _NBUF == 0

    @functools.partial(
        pl.kernel,
        mesh=mesh,
        compiler_params=pltpu.CompilerParams(needs_layout_passes=False,
                                             use_tc_tiling_on_sc=False),
        out_type=jax.ShapeDtypeStruct((B, Lseq, E), jnp.float32),
        scratch_types=[
            pltpu.VMEM((bpw, Lseq), jnp.int32),
            pltpu.VMEM((_NBUF, Lseq, E), jnp.float32),
            pltpu.VMEM((_NBUF, Lseq, E), jnp.float32),
            pltpu.VMEM((E,), jnp.float32),
            pltpu.VMEM((E,), jnp.float32),
        ] + [pltpu.SemaphoreType.DMA] * (2 * _NBUF),
    )
    def sc_kernel(ids_hbm, tab_hbm, gamma_hbm, beta_hbm, out_hbm,
                  idx_v, rows_v, res_v, g_v, b_v, *sems):
        gsem = sems[:_NBUF]
        wsem = sems[_NBUF:]
        wid = lax.axis_index("s") * _NC + lax.axis_index("c")
        b0 = wid * bpw
        pltpu.sync_copy(ids_hbm.at[pl.ds(b0, bpw)], idx_v)
        pltpu.sync_copy(gamma_hbm, g_v)
        pltpu.sync_copy(beta_hbm, b_v)
        inv_e = jnp.float32(1.0 / E)

        def start_gather(b, j):
            for off, sz in splits:
                pltpu.async_copy(
                    tab_hbm.at[idx_v.at[j, pl.ds(off, sz)]],
                    rows_v.at[b, pl.ds(off, sz)], gsem[b])

        def wait_gather(b, j):
            for off, sz in splits:
                pltpu.make_async_copy(
                    tab_hbm.at[idx_v.at[j, pl.ds(off, sz)]],
                    rows_v.at[b, pl.ds(off, sz)], gsem[b]).wait()

        def start_write(b, j):
            pltpu.async_copy(res_v.at[b], out_hbm.at[b0 + j], wsem[b])

        def wait_write(b, j):
            pltpu.make_async_copy(res_v.at[b], out_hbm.at[b0 + j],
                                  wsem[b]).wait()

        nq = E // _L
        gvs = [g_v[pl.ds(_L * i, _L)] for i in range(nq)]
        bvs = [b_v[pl.ds(_L * i, _L)] for i in range(nq)]

        def compute(b):
            rows = rows_v.at[b]
            res = res_v.at[b]

            def row_body(r, carry):
                vs = [rows[r, pl.ds(_L * i, _L)] for i in range(nq)]
                s = vs[0]
                q = vs[0] * vs[0]
                for i in range(1, nq):
                    s = s + vs[i]
                    q = q + vs[i] * vs[i]
                tot = jnp.full((_L,), jnp.sum(s), dtype=jnp.float32)
                qtot = jnp.full((_L,), jnp.sum(q), dtype=jnp.float32)
                mean = tot * inv_e
                var = jnp.maximum(qtot * inv_e - mean * mean,
                                  0.0) + jnp.float32(_EPS)
                rinv = _rsqrt16(var)
                for i in range(nq):
                    res[r, pl.ds(_L * i, _L)] = (
                        (vs[i] - mean) * (rinv * gvs[i]) + bvs[i])
                return carry

            lax.fori_loop(0, Lseq, row_body, 0, unroll=4)

        for b in range(_NBUF):
            start_gather(b, b)

        def outer(s, carry):
            for b in range(_NBUF):
                j = s * _NBUF + b
                wait_gather(b, j)

                @pl.when(s > 0)
                def _():
                    wait_write(b, j - _NBUF)

                compute(b)
                start_gather(b, j + _NBUF)
                start_write(b, j)
            return carry

        n_steady = bpw // _NBUF - 1
        lax.fori_loop(0, n_steady, outer, 0)

        for b in range(_NBUF):
            j = n_steady * _NBUF + b
            wait_gather(b, j)
            wait_write(b, j - _NBUF)
            compute(b)
            start_write(b, j)
        for b in range(_NBUF):
            wait_write(b, n_steady * _NBUF + b)

    return sc_kernel


def kernel(input_ids, table, gamma, beta):
    B, Lseq = input_ids.shape
    V, E = table.shape
    return _make_sc_kernel(B, Lseq, E)(input_ids, table, gamma, beta)
